# baseline (speedup 1.0000x reference)
import functools

import jax
import jax.numpy as jnp
from jax import lax
from jax.experimental import pallas as pl
from jax.experimental.pallas import tpu as pltpu
from jax.experimental.pallas import tpu_sc as plsc

N_NODES = 10000
N_EDGES = 320000
D_IN = 128
D_HID = 256
D_MID = 64
D_OUT = 10
N_GRAPHS = 64

_NC = 2
_NS = 16
_NW = _NC * _NS
_L = 16

_CHUNK = 128
_NCH = 80
_E_PAD = _NW * _NCH * _CHUNK
_N_PAD = 10240
_RPS = _N_PAD // _NS
_ZR = 40
_Q = 16


def _make_sc_aggregate(dg, ds):
  mesh = plsc.VectorSubcoreMesh(core_axis_name="c", subcore_axis_name="s")
  compact = ds < dg

  scratch = [
      pltpu.VMEM((_Q, _CHUNK), jnp.int32),
      pltpu.VMEM((_Q, _CHUNK), jnp.int32),
      pltpu.VMEM((_Q, _CHUNK), jnp.float32),
      pltpu.VMEM((_CHUNK, dg), jnp.float32),
      pltpu.VMEM((_CHUNK, dg), jnp.float32),
      pltpu.VMEM((_ZR, ds), jnp.float32),
      pltpu.VMEM_SHARED((_N_PAD, ds), jnp.float32),
      pltpu.SemaphoreType.DMA,
      pltpu.SemaphoreType.DMA,
      pltpu.SemaphoreType.DMA,
      pltpu.SemaphoreType.DMA,
      pltpu.SemaphoreType.DMA,
  ]
  if compact:
    scratch[5:5] = [
        pltpu.VMEM((_CHUNK, ds), jnp.float32),
        pltpu.VMEM((_CHUNK, ds), jnp.float32),
    ]

  @functools.partial(
      pl.kernel,
      out_type=jax.ShapeDtypeStruct((_NC, _N_PAD, ds), jnp.float32),
      mesh=mesh,
      scratch_types=scratch,
  )
  def agg(x_hbm, src_hbm, dst_hbm, w_hbm, out_hbm, src_v, dst_v, w_v,
          g0, g1, *rest):
    if compact:
      s0, s1, zero_v, acc, gs0, gs1, ss0, ss1, xsem = rest
    else:
      zero_v, acc, gs0, gs1, ss0, ss1, xsem = rest
      s0, s1 = g0, g1
    gbuf = (g0, g1)
    sbuf = (s0, s1)
    gsem = (gs0, gs1)
    ssem = (ss0, ss1)

    c = lax.axis_index("c")
    s = lax.axis_index("s")
    wid = c * _NS + s

    scope_zero = jax.named_scope("agg_zero")
    scope_zero.__enter__()
    zv = jnp.zeros((_L,), jnp.float32)

    def zrow(i, carry):
      for j in range(ds // _L):
        zero_v[i, pl.ds(j * _L, _L)] = zv
      return carry

    lax.fori_loop(0, _ZR, zrow, 0)

    def zcopy(r, carry):
      pltpu.sync_copy(zero_v, acc.at[pl.ds(s * _RPS + r * _ZR, _ZR), :])
      return carry

    lax.fori_loop(0, _RPS // _ZR, zcopy, 0)
    scope_zero.__exit__(None, None, None)
    plsc.subcore_barrier()

    def multiply(buf, ch):
      gv = gbuf[buf]
      sv = sbuf[buf]

      def emul(g, c2):
        wvec = w_v[ch, pl.ds(g * _L, _L)]
        for l in range(_L):
          wt = wvec[l]
          e = g * _L + l
          for j in range(ds // _L):
            sv[e, pl.ds(j * _L, _L)] = gv[e, pl.ds(j * _L, _L)] * wt
        return c2

      lax.fori_loop(0, _CHUNK // _L, emul, 0)

    with jax.named_scope("agg_main"):
      def seg_body(seg, carry):
        pltpu.sync_copy(src_hbm.at[wid, pl.ds(seg * _Q, _Q)], src_v)
        pltpu.sync_copy(dst_hbm.at[wid, pl.ds(seg * _Q, _Q)], dst_v)
        pltpu.sync_copy(w_hbm.at[wid, pl.ds(seg * _Q, _Q)], w_v)

        def pair(p, c1):
          h0 = pltpu.async_copy(x_hbm.at[src_v.at[2 * p]], g0, gsem[0])

          @pl.when(p > 0)
          def _():
            pltpu.make_async_copy(g1, acc.at[dst_v.at[0]], ssem[1]).wait()

          h1 = pltpu.async_copy(x_hbm.at[src_v.at[2 * p + 1]], g1, gsem[1])
          h0.wait()
          multiply(0, 2 * p)
          sc0 = pltpu.async_copy(g0, acc.at[dst_v.at[2 * p]], ssem[0],
                                 add=True)
          h1.wait()
          multiply(1, 2 * p + 1)
          sc0.wait()
          pltpu.async_copy(g1, acc.at[dst_v.at[2 * p + 1]], ssem[1], add=True)
          return c1

        lax.fori_loop(0, _Q // 2, pair, 0)
        pltpu.make_async_copy(g1, acc.at[dst_v.at[0]], ssem[1]).wait()
        return carry

      lax.fori_loop(0, _NCH // _Q, seg_body, 0)
    plsc.subcore_barrier()

    with jax.named_scope("agg_out"):
      def ocopy(r, carry):
        pltpu.sync_copy(acc.at[pl.ds(s * _RPS + r * _ZR, _ZR), :],
                        out_hbm.at[c, pl.ds(s * _RPS + r * _ZR, _ZR), :])
        return carry

      lax.fori_loop(0, _RPS // _ZR, ocopy, 0)

  return agg


_sc_agg_128 = _make_sc_aggregate(D_IN, D_IN)


def _copy_body(x_ref, o_ref):
  o_ref[...] = x_ref[...]


def _tc_copy(x):
  return pl.pallas_call(
      _copy_body,
      grid=(5,),
      in_specs=[pl.BlockSpec((2000, D_IN), lambda i: (i, 0))],
      out_specs=pl.BlockSpec((2000, D_IN), lambda i: (i, 0)),
      out_shape=jax.ShapeDtypeStruct((N_NODES, D_IN), jnp.float32),
  )(x)


_BS1 = 2000
_NB1 = N_NODES // _BS1


def _p1_body(agg_ref, x_ref, w1r_ref, b1_ref, w1o_ref, w2r_ref, w2o_ref,
             p2_ref, r2_ref):
  a = agg_ref[0] + agg_ref[1]
  h = lax.dot_general(a, w1r_ref[...], (((1,), (0,)), ((), ())),
                      preferred_element_type=jnp.float32)
  h = h + b1_ref[...]
  h = h + lax.dot_general(x_ref[...], w1o_ref[...], (((1,), (0,)), ((), ())),
                          preferred_element_type=jnp.float32)
  h = jnp.maximum(h, 0.0)
  p2 = lax.dot_general(h, w2r_ref[...], (((1,), (0,)), ((), ())),
                       preferred_element_type=jnp.float32)
  p2_ref[...] = jnp.concatenate([p2, jnp.zeros_like(p2)], axis=1)
  r2_ref[...] = lax.dot_general(h, w2o_ref[...], (((1,), (0,)), ((), ())),
                                preferred_element_type=jnp.float32)


def _tc_phase1(agg1p, x, w1r_t, b1, w1o_t, w2r_t, w2o_t):
  return pl.pallas_call(
      _p1_body,
      grid=(_NB1,),
      in_specs=[
          pl.BlockSpec((_NC, _BS1, D_IN), lambda i: (0, i, 0)),
          pl.BlockSpec((_BS1, D_IN), lambda i: (i, 0)),
          pl.BlockSpec((D_IN, D_HID), lambda i: (0, 0)),
          pl.BlockSpec((1, D_HID), lambda i: (0, 0)),
          pl.BlockSpec((D_IN, D_HID), lambda i: (0, 0)),
          pl.BlockSpec((D_HID, D_MID), lambda i: (0, 0)),
          pl.BlockSpec((D_HID, D_MID), lambda i: (0, 0)),
      ],
      out_specs=[
          pl.BlockSpec((_BS1, D_IN), lambda i: (i, 0)),
          pl.BlockSpec((_BS1, D_MID), lambda i: (i, 0)),
      ],
      out_shape=[
          jax.ShapeDtypeStruct((N_NODES, D_IN), jnp.float32),
          jax.ShapeDtypeStruct((N_NODES, D_MID), jnp.float32),
      ],
  )(agg1p, x, w1r_t, b1, w1o_t, w2r_t, w2o_t)


def _p2_body(agg_ref, r2_ref, b2_ref, batch_ref, wc_ref, bc_ref, out_ref):
  agg = agg_ref[0, :, :D_MID] + agg_ref[1, :, :D_MID]
  h2 = agg + b2_ref[...] + r2_ref[...]
  h2 = jnp.maximum(h2, 0.0)
  gids = lax.broadcasted_iota(jnp.int32, (N_GRAPHS, 1), 0)
  onehot_t = (gids == batch_ref[...]).astype(jnp.float32)
  sums = lax.dot_general(onehot_t, h2, (((1,), (0,)), ((), ())),
                         preferred_element_type=jnp.float32)
  counts = jnp.sum(onehot_t, axis=1, keepdims=True)
  pooled = sums / jnp.maximum(counts, 1.0)
  out_ref[...] = lax.dot_general(pooled, wc_ref[...], (((1,), (0,)), ((), ())),
                                 preferred_element_type=jnp.float32) + bc_ref[...]


def _tc_phase2(agg2p, r2, b2, batch_row, wc_t, bc):
  return pl.pallas_call(
      _p2_body,
      grid=(1,),
      in_specs=[
          pl.BlockSpec((_NC, N_NODES, D_IN), lambda i: (0, 0, 0)),
          pl.BlockSpec((N_NODES, D_MID), lambda i: (0, 0)),
          pl.BlockSpec((1, D_MID), lambda i: (0, 0)),
          pl.BlockSpec((1, N_NODES), lambda i: (0, 0)),
          pl.BlockSpec((D_MID, D_OUT), lambda i: (0, 0)),
          pl.BlockSpec((1, D_OUT), lambda i: (0, 0)),
      ],
      out_specs=pl.BlockSpec((N_GRAPHS, D_OUT), lambda i: (0, 0)),
      out_shape=jax.ShapeDtypeStruct((N_GRAPHS, D_OUT), jnp.float32),
  )(agg2p, r2, b2, batch_row, wc_t, bc)


def kernel(x, edge_index, edge_weight, batch, W1_rel, b1_rel, W1_root,
           W2_rel, b2_rel, W2_root, Wc, bc):
  x = x.astype(jnp.float32)
  edge_weight = edge_weight.astype(jnp.float32)

  pad = _E_PAD - N_EDGES
  src = jnp.concatenate(
      [edge_index[0].astype(jnp.int32), jnp.zeros((pad,), jnp.int32)])
  dst = jnp.concatenate(
      [edge_index[1].astype(jnp.int32), jnp.zeros((pad,), jnp.int32)])
  w = jnp.concatenate([edge_weight, jnp.zeros((pad,), jnp.float32)])
  src = src.reshape(_NW, _NCH, _CHUNK)
  dst = dst.reshape(_NW, _NCH, _CHUNK)
  w = w.reshape(_NW, _NCH, _CHUNK)

  xc = _tc_copy(x)
  agg1p = _sc_agg_128(xc, src, dst, w)
  p2, r2 = _tc_phase1(agg1p, x, W1_rel.T, b1_rel.reshape(1, D_HID),
                      W1_root.T, W2_rel.T, W2_root.T)
  agg2p = _sc_agg_128(p2, src, dst, w)
  out = _tc_phase2(agg2p, r2, b2_rel.reshape(1, D_MID),
                   batch.astype(jnp.int32).reshape(1, N_NODES), Wc.T, bc.reshape(1, D_OUT))
  return out


# speedup vs baseline: 1.0102x; 1.0102x over previous
import functools

import jax
import jax.numpy as jnp
from jax import lax
from jax.experimental import pallas as pl
from jax.experimental.pallas import tpu as pltpu
from jax.experimental.pallas import tpu_sc as plsc

N_NODES = 10000
N_EDGES = 320000
D_IN = 128
D_HID = 256
D_MID = 64
D_OUT = 10
N_GRAPHS = 64

_NC = 2
_NS = 16
_NW = _NC * _NS
_L = 16

_CHUNK = 128
_NCH = 80
_E_PAD = _NW * _NCH * _CHUNK
_N_PAD = 10240
_RPS = _N_PAD // _NS
_ZR = 40
_Q = 16


def _make_sc_aggregate(dg, ds):
  mesh = plsc.VectorSubcoreMesh(core_axis_name="c", subcore_axis_name="s")
  compact = ds < dg

  scratch = [
      pltpu.VMEM((_Q, _CHUNK), jnp.int32),
      pltpu.VMEM((_Q, _CHUNK), jnp.int32),
      pltpu.VMEM((_Q, _CHUNK), jnp.float32),
      pltpu.VMEM((_CHUNK, dg), jnp.float32),
      pltpu.VMEM((_CHUNK, dg), jnp.float32),
      pltpu.VMEM((_ZR, ds), jnp.float32),
      pltpu.VMEM_SHARED((_N_PAD, ds), jnp.float32),
      pltpu.SemaphoreType.DMA,
      pltpu.SemaphoreType.DMA,
      pltpu.SemaphoreType.DMA,
      pltpu.SemaphoreType.DMA,
      pltpu.SemaphoreType.DMA,
  ]
  if compact:
    scratch[5:5] = [
        pltpu.VMEM((_CHUNK, ds), jnp.float32),
        pltpu.VMEM((_CHUNK, ds), jnp.float32),
    ]

  @functools.partial(
      pl.kernel,
      out_type=jax.ShapeDtypeStruct((_NC, _N_PAD, ds), jnp.float32),
      mesh=mesh,
      scratch_types=scratch,
  )
  def agg(x_hbm, src_hbm, dst_hbm, w_hbm, out_hbm, src_v, dst_v, w_v,
          g0, g1, *rest):
    if compact:
      s0, s1, zero_v, acc, gs0, gs1, ss0, ss1, xsem = rest
    else:
      zero_v, acc, gs0, gs1, ss0, ss1, xsem = rest
      s0, s1 = g0, g1
    gbuf = (g0, g1)
    sbuf = (s0, s1)
    gsem = (gs0, gs1)
    ssem = (ss0, ss1)

    c = lax.axis_index("c")
    s = lax.axis_index("s")
    wid = c * _NS + s

    scope_zero = jax.named_scope("agg_zero")
    scope_zero.__enter__()
    zv = jnp.zeros((_L,), jnp.float32)

    def zrow(i, carry):
      for j in range(ds // _L):
        zero_v[i, pl.ds(j * _L, _L)] = zv
      return carry

    lax.fori_loop(0, _ZR, zrow, 0)
    handles = [
        pltpu.async_copy(
            zero_v, acc.at[pl.ds(s * _RPS + r * _ZR, _ZR), :], xsem)
        for r in range(_RPS // _ZR)
    ]
    for h in handles:
      h.wait()
    scope_zero.__exit__(None, None, None)
    plsc.subcore_barrier()

    def multiply(buf, ch):
      gv = gbuf[buf]
      sv = sbuf[buf]

      def emul(g, c2):
        wvec = w_v[ch, pl.ds(g * _L, _L)]
        for l in range(_L):
          wt = wvec[l]
          e = g * _L + l
          for j in range(ds // _L):
            sv[e, pl.ds(j * _L, _L)] = gv[e, pl.ds(j * _L, _L)] * wt
        return c2

      lax.fori_loop(0, _CHUNK // _L, emul, 0)

    with jax.named_scope("agg_main"):
      def seg_body(seg, carry):
        pltpu.sync_copy(src_hbm.at[wid, pl.ds(seg * _Q, _Q)], src_v)
        pltpu.sync_copy(dst_hbm.at[wid, pl.ds(seg * _Q, _Q)], dst_v)
        pltpu.sync_copy(w_hbm.at[wid, pl.ds(seg * _Q, _Q)], w_v)

        def pair(p, c1):
          h0 = pltpu.async_copy(x_hbm.at[src_v.at[2 * p]], g0, gsem[0])

          @pl.when(p > 0)
          def _():
            pltpu.make_async_copy(g1, acc.at[dst_v.at[0]], ssem[1]).wait()

          h1 = pltpu.async_copy(x_hbm.at[src_v.at[2 * p + 1]], g1, gsem[1])
          h0.wait()
          multiply(0, 2 * p)
          sc0 = pltpu.async_copy(g0, acc.at[dst_v.at[2 * p]], ssem[0],
                                 add=True)
          h1.wait()
          multiply(1, 2 * p + 1)
          sc0.wait()
          pltpu.async_copy(g1, acc.at[dst_v.at[2 * p + 1]], ssem[1], add=True)
          return c1

        lax.fori_loop(0, _Q // 2, pair, 0)
        pltpu.make_async_copy(g1, acc.at[dst_v.at[0]], ssem[1]).wait()
        return carry

      lax.fori_loop(0, _NCH // _Q, seg_body, 0)
    plsc.subcore_barrier()

    with jax.named_scope("agg_out"):
      handles = [
          pltpu.async_copy(
              acc.at[pl.ds(s * _RPS + r * _ZR, _ZR), :],
              out_hbm.at[c, pl.ds(s * _RPS + r * _ZR, _ZR), :], xsem)
          for r in range(_RPS // _ZR)
      ]
      for h in handles:
        h.wait()

  return agg


_sc_agg_128 = _make_sc_aggregate(D_IN, D_IN)


def _copy_body(x_ref, o_ref):
  o_ref[...] = x_ref[...]


def _tc_copy(x):
  return pl.pallas_call(
      _copy_body,
      grid=(5,),
      in_specs=[pl.BlockSpec((2000, D_IN), lambda i: (i, 0))],
      out_specs=pl.BlockSpec((2000, D_IN), lambda i: (i, 0)),
      out_shape=jax.ShapeDtypeStruct((N_NODES, D_IN), jnp.float32),
  )(x)


_BS1 = 2000
_NB1 = N_NODES // _BS1


def _p1_body(agg_ref, x_ref, w1r_ref, b1_ref, w1o_ref, w2r_ref, w2o_ref,
             p2_ref, r2_ref):
  a = agg_ref[0] + agg_ref[1]
  h = lax.dot_general(a, w1r_ref[...], (((1,), (0,)), ((), ())),
                      preferred_element_type=jnp.float32)
  h = h + b1_ref[...]
  h = h + lax.dot_general(x_ref[...], w1o_ref[...], (((1,), (0,)), ((), ())),
                          preferred_element_type=jnp.float32)
  h = jnp.maximum(h, 0.0)
  p2 = lax.dot_general(h, w2r_ref[...], (((1,), (0,)), ((), ())),
                       preferred_element_type=jnp.float32)
  p2_ref[...] = jnp.concatenate([p2, jnp.zeros_like(p2)], axis=1)
  r2_ref[...] = lax.dot_general(h, w2o_ref[...], (((1,), (0,)), ((), ())),
                                preferred_element_type=jnp.float32)


def _tc_phase1(agg1p, x, w1r_t, b1, w1o_t, w2r_t, w2o_t):
  return pl.pallas_call(
      _p1_body,
      grid=(_NB1,),
      in_specs=[
          pl.BlockSpec((_NC, _BS1, D_IN), lambda i: (0, i, 0)),
          pl.BlockSpec((_BS1, D_IN), lambda i: (i, 0)),
          pl.BlockSpec((D_IN, D_HID), lambda i: (0, 0)),
          pl.BlockSpec((1, D_HID), lambda i: (0, 0)),
          pl.BlockSpec((D_IN, D_HID), lambda i: (0, 0)),
          pl.BlockSpec((D_HID, D_MID), lambda i: (0, 0)),
          pl.BlockSpec((D_HID, D_MID), lambda i: (0, 0)),
      ],
      out_specs=[
          pl.BlockSpec((_BS1, D_IN), lambda i: (i, 0)),
          pl.BlockSpec((_BS1, D_MID), lambda i: (i, 0)),
      ],
      out_shape=[
          jax.ShapeDtypeStruct((N_NODES, D_IN), jnp.float32),
          jax.ShapeDtypeStruct((N_NODES, D_MID), jnp.float32),
      ],
  )(agg1p, x, w1r_t, b1, w1o_t, w2r_t, w2o_t)


def _p2_body(agg_ref, r2_ref, b2_ref, batch_ref, wc_ref, bc_ref, out_ref):
  agg = agg_ref[0, :, :D_MID] + agg_ref[1, :, :D_MID]
  h2 = agg + b2_ref[...] + r2_ref[...]
  h2 = jnp.maximum(h2, 0.0)
  gids = lax.broadcasted_iota(jnp.int32, (N_GRAPHS, 1), 0)
  onehot_t = (gids == batch_ref[...]).astype(jnp.float32)
  sums = lax.dot_general(onehot_t, h2, (((1,), (0,)), ((), ())),
                         preferred_element_type=jnp.float32)
  counts = jnp.sum(onehot_t, axis=1, keepdims=True)
  pooled = sums / jnp.maximum(counts, 1.0)
  out_ref[...] = lax.dot_general(pooled, wc_ref[...], (((1,), (0,)), ((), ())),
                                 preferred_element_type=jnp.float32) + bc_ref[...]


def _tc_phase2(agg2p, r2, b2, batch_row, wc_t, bc):
  return pl.pallas_call(
      _p2_body,
      grid=(1,),
      in_specs=[
          pl.BlockSpec((_NC, N_NODES, D_IN), lambda i: (0, 0, 0)),
          pl.BlockSpec((N_NODES, D_MID), lambda i: (0, 0)),
          pl.BlockSpec((1, D_MID), lambda i: (0, 0)),
          pl.BlockSpec((1, N_NODES), lambda i: (0, 0)),
          pl.BlockSpec((D_MID, D_OUT), lambda i: (0, 0)),
          pl.BlockSpec((1, D_OUT), lambda i: (0, 0)),
      ],
      out_specs=pl.BlockSpec((N_GRAPHS, D_OUT), lambda i: (0, 0)),
      out_shape=jax.ShapeDtypeStruct((N_GRAPHS, D_OUT), jnp.float32),
  )(agg2p, r2, b2, batch_row, wc_t, bc)


def kernel(x, edge_index, edge_weight, batch, W1_rel, b1_rel, W1_root,
           W2_rel, b2_rel, W2_root, Wc, bc):
  x = x.astype(jnp.float32)
  edge_weight = edge_weight.astype(jnp.float32)

  pad = _E_PAD - N_EDGES
  src = jnp.concatenate(
      [edge_index[0].astype(jnp.int32), jnp.zeros((pad,), jnp.int32)])
  dst = jnp.concatenate(
      [edge_index[1].astype(jnp.int32), jnp.zeros((pad,), jnp.int32)])
  w = jnp.concatenate([edge_weight, jnp.zeros((pad,), jnp.float32)])
  src = src.reshape(_NW, _NCH, _CHUNK)
  dst = dst.reshape(_NW, _NCH, _CHUNK)
  w = w.reshape(_NW, _NCH, _CHUNK)

  xc = _tc_copy(x)
  agg1p = _sc_agg_128(xc, src, dst, w)
  p2, r2 = _tc_phase1(agg1p, x, W1_rel.T, b1_rel.reshape(1, D_HID),
                      W1_root.T, W2_rel.T, W2_root.T)
  agg2p = _sc_agg_128(p2, src, dst, w)
  out = _tc_phase2(agg2p, r2, b2_rel.reshape(1, D_MID),
                   batch.astype(jnp.int32).reshape(1, N_NODES), Wc.T, bc.reshape(1, D_OUT))
  return out
